# Initial kernel scaffold; baseline (speedup 1.0000x reference)
import jax
import jax.numpy as jnp
from jax.experimental import pallas as pl


def kernel(X, r, mus):
    raise NotImplementedError("write your pallas kernel here")


# speedup vs baseline: 1.1080x; 1.1080x over previous
import jax
import jax.numpy as jnp
from jax.experimental import pallas as pl
from jax.experimental.pallas import tpu as pltpu

_BN = 2048


def _loss_body(x_ref, r_ref, mu_ref, o_ref):
    x = x_ref[...]
    mu = mu_ref[...]
    x2 = jnp.sum(x * x, axis=1, keepdims=True)
    mu2 = jnp.sum(mu * mu, axis=1, keepdims=True).T
    cross = jax.lax.dot_general(
        x, mu,
        dimension_numbers=(((1,), (1,)), ((), ())),
        preferred_element_type=jnp.float32,
    )
    dist2 = jnp.maximum(x2 + mu2 - 2.0 * cross, 0.0)
    o_ref[0, 0, :] = jnp.sum(r_ref[...] * dist2, axis=0)


def kernel(X, r, mus):
    n, d = X.shape
    m = mus.shape[0]
    g = n // _BN
    partials = pl.pallas_call(
        _loss_body,
        grid=(g,),
        in_specs=[
            pl.BlockSpec((_BN, d), lambda i: (i, 0)),
            pl.BlockSpec((_BN, m), lambda i: (i, 0)),
            pl.BlockSpec((m, d), lambda i: (0, 0)),
        ],
        out_specs=pl.BlockSpec((1, 1, m), lambda i: (i, 0, 0)),
        out_shape=jax.ShapeDtypeStruct((g, 1, m), jnp.float32),
        compiler_params=pltpu.CompilerParams(
            dimension_semantics=("parallel",),
        ),
    )(X, r, mus)
    return jnp.sum(partials) / n
